# Initial kernel scaffold; baseline (speedup 1.0000x reference)
import jax
import jax.numpy as jnp
from jax.experimental import pallas as pl


def kernel(x0, x1, edge_index, edge_weight, W1_0, b1_0, W2_0, b2_0, W1_1, b1_1, W2_1, b2_1, A1, a1b, A2, A3):
    raise NotImplementedError("write your pallas kernel here")


# speedup vs baseline: 6.1869x; 6.1869x over previous
import functools

import jax
import jax.numpy as jnp
from jax import lax
from jax.experimental import pallas as pl
from jax.experimental.pallas import tpu as pltpu
from jax.experimental.pallas import tpu_sc as plsc

N = 10000
E = 320000
D = 128
H = 32
C = 16

NC = 2
NS = 16
NW = NC * NS
EPW = E // NW
CH = 80
NCHUNK = EPW // CH
NPAD = 10240
RPT = NPAD // NS


def _make_spmm(F):
    fv = F // 16
    mesh = plsc.VectorSubcoreMesh(
        core_axis_name="c", subcore_axis_name="s", num_cores=NC, num_subcores=NS
    )

    @functools.partial(
        pl.kernel,
        out_type=jax.ShapeDtypeStruct((NC * NPAD, F), jnp.float32),
        mesh=mesh,
        compiler_params=pltpu.CompilerParams(use_tc_tiling_on_sc=False),
        scratch_types=[
            pltpu.VMEM((CH,), jnp.int32),
            pltpu.VMEM((CH,), jnp.int32),
            pltpu.VMEM((CH,), jnp.float32),
            pltpu.VMEM((CH, F), jnp.float32),
            pltpu.VMEM_SHARED((NPAD, F), jnp.float32),
            pltpu.SemaphoreType.DMA,
        ],
    )
    def spmm(src_hbm, dst_hbm, ew_hbm, sup_hbm, out_hbm, src_v, dst_v, ew_v,
             rows_v, acc_sh, sem):
        cid = lax.axis_index("c")
        sid = lax.axis_index("s")
        wid = cid * NS + sid

        def zero_row(e, carry):
            for k in range(fv):
                rows_v[e, pl.ds(16 * k, 16)] = jnp.zeros((16,), jnp.float32)
            return carry

        lax.fori_loop(0, CH, zero_row, 0, unroll=4)
        for r in range(RPT // CH):
            pltpu.sync_copy(rows_v, acc_sh.at[pl.ds(sid * RPT + r * CH, CH)])
        plsc.subcore_barrier()

        def chunk(j, carry):
            base = wid * EPW + j * CH
            pltpu.sync_copy(src_hbm.at[pl.ds(base, CH)], src_v)
            pltpu.sync_copy(dst_hbm.at[pl.ds(base, CH)], dst_v)
            pltpu.sync_copy(ew_hbm.at[pl.ds(base, CH)], ew_v)
            pltpu.async_copy(sup_hbm.at[src_v], rows_v, sem).wait()

            def scale(jj, c2):
                wv = ew_v[pl.ds(16 * jj, 16)]
                for l in range(16):
                    w = wv[l]
                    e = 16 * jj + l
                    for k in range(fv):
                        rows_v[e, pl.ds(16 * k, 16)] = (
                            rows_v[e, pl.ds(16 * k, 16)] * w
                        )
                return c2

            lax.fori_loop(0, CH // 16, scale, 0)
            pltpu.sync_copy(rows_v, acc_sh.at[dst_v], add=True)
            return carry

        lax.fori_loop(0, NCHUNK, chunk, 0)
        plsc.subcore_barrier()

        pltpu.sync_copy(
            acc_sh.at[pl.ds(sid * RPT, RPT)],
            out_hbm.at[pl.ds(cid * NPAD + sid * RPT, RPT)],
        )

    return spmm


_spmm64 = _make_spmm(64)
_spmm32 = _make_spmm(32)

_RB = 1000


def _tc1_body(x0_ref, x1_ref, w10_ref, w11_ref, out_ref):
    s0 = jnp.dot(x0_ref[...], w10_ref[...], preferred_element_type=jnp.float32)
    s1 = jnp.dot(x1_ref[...], w11_ref[...], preferred_element_type=jnp.float32)
    out_ref[...] = jnp.concatenate([s0, s1], axis=1)


def _tc1(x0, x1, W1_0, W1_1):
    return pl.pallas_call(
        _tc1_body,
        grid=(N // _RB,),
        in_specs=[
            pl.BlockSpec((_RB, D), lambda i: (i, 0)),
            pl.BlockSpec((_RB, D), lambda i: (i, 0)),
            pl.BlockSpec((D, H), lambda i: (0, 0)),
            pl.BlockSpec((D, H), lambda i: (0, 0)),
        ],
        out_specs=pl.BlockSpec((_RB, 2 * H), lambda i: (i, 0)),
        out_shape=jax.ShapeDtypeStruct((N, 2 * H), jnp.float32),
    )(x0, x1, W1_0, W1_1)


def _tc2_body(p0_ref, p1_ref, b10_ref, b11_ref, w20_ref, w21_ref, out_ref):
    agg = p0_ref[...] + p1_ref[...]
    h0 = jax.nn.relu(agg[:, :H] + b10_ref[...])
    h1 = jax.nn.relu(agg[:, H:] + b11_ref[...])
    o0 = jnp.dot(h0, w20_ref[...], preferred_element_type=jnp.float32)
    o1 = jnp.dot(h1, w21_ref[...], preferred_element_type=jnp.float32)
    out_ref[...] = jnp.concatenate([o0, o1], axis=1)


def _tc2(p0, p1, b1_0, b1_1, W2_0, W2_1):
    return pl.pallas_call(
        _tc2_body,
        grid=(N // _RB,),
        in_specs=[
            pl.BlockSpec((_RB, 2 * H), lambda i: (i, 0)),
            pl.BlockSpec((_RB, 2 * H), lambda i: (i, 0)),
            pl.BlockSpec((1, H), lambda i: (0, 0)),
            pl.BlockSpec((1, H), lambda i: (0, 0)),
            pl.BlockSpec((H, C), lambda i: (0, 0)),
            pl.BlockSpec((H, C), lambda i: (0, 0)),
        ],
        out_specs=pl.BlockSpec((_RB, 2 * C), lambda i: (i, 0)),
        out_shape=jax.ShapeDtypeStruct((N, 2 * C), jnp.float32),
    )(p0, p1, b1_0.reshape(1, H), b1_1.reshape(1, H), W2_0, W2_1)


def _log_softmax(x):
    m = jnp.max(x, axis=1, keepdims=True)
    s = x - m
    return s - jnp.log(jnp.sum(jnp.exp(s), axis=1, keepdims=True))


def _tc3_body(p0_ref, p1_ref, b20_ref, b21_ref, a1_ref, a1b_ref, a2_ref,
              a3_ref, out_ref):
    agg = p0_ref[...] + p1_ref[...]
    z0 = _log_softmax(agg[:, :C] + b20_ref[...])
    z1 = _log_softmax(agg[:, C:] + b21_ref[...])
    a1 = a1_ref[...]
    a1b = a1b_ref[...]
    a2 = a2_ref[...]
    a3 = a3_ref[...]
    h0 = jnp.tanh(jnp.dot(z0, a1, preferred_element_type=jnp.float32) + a1b)
    h1 = jnp.tanh(jnp.dot(z1, a1, preferred_element_type=jnp.float32) + a1b)
    g0 = jnp.tanh(jnp.dot(h0, a2, preferred_element_type=jnp.float32))
    g1 = jnp.tanh(jnp.dot(h1, a2, preferred_element_type=jnp.float32))
    w0 = jnp.sum(g0 * a3, axis=1, keepdims=True)
    w1 = jnp.sum(g1 * a3, axis=1, keepdims=True)
    m = jnp.maximum(w0, w1)
    s0 = jnp.exp(w0 - m)
    s1 = jnp.exp(w1 - m)
    out_ref[...] = (s0 * z0 + s1 * z1) / (s0 + s1)


def _tc3(p0, p1, b2_0, b2_1, A1, a1b, A2, A3):
    FH = A1.shape[1]
    return pl.pallas_call(
        _tc3_body,
        grid=(N // _RB,),
        in_specs=[
            pl.BlockSpec((_RB, 2 * C), lambda i: (i, 0)),
            pl.BlockSpec((_RB, 2 * C), lambda i: (i, 0)),
            pl.BlockSpec((1, C), lambda i: (0, 0)),
            pl.BlockSpec((1, C), lambda i: (0, 0)),
            pl.BlockSpec((C, FH), lambda i: (0, 0)),
            pl.BlockSpec((1, FH), lambda i: (0, 0)),
            pl.BlockSpec((FH, 32), lambda i: (0, 0)),
            pl.BlockSpec((1, 32), lambda i: (0, 0)),
        ],
        out_specs=pl.BlockSpec((_RB, C), lambda i: (i, 0)),
        out_shape=jax.ShapeDtypeStruct((N, C), jnp.float32),
    )(p0, p1, b2_0.reshape(1, C), b2_1.reshape(1, C), A1,
      a1b.reshape(1, FH), A2, A3.reshape(1, 32))


def kernel(x0, x1, edge_index, edge_weight, W1_0, b1_0, W2_0, b2_0, W1_1,
           b1_1, W2_1, b2_1, A1, a1b, A2, A3):
    src = edge_index[1].astype(jnp.int32)
    dst = edge_index[0].astype(jnp.int32)
    ew = edge_weight.astype(jnp.float32)

    sup = _tc1(x0, x1, W1_0, W1_1)
    parts = _spmm64(src, dst, ew, sup)
    h = _tc2(parts[:N], parts[NPAD:NPAD + N], b1_0, b1_1, W2_0, W2_1)
    parts2 = _spmm32(src, dst, ew, h)
    return _tc3(parts2[:N], parts2[NPAD:NPAD + N], b2_0, b2_1, A1, a1b, A2,
                A3)


# speedup vs baseline: 10.7813x; 1.7426x over previous
import functools

import jax
import jax.numpy as jnp
from jax import lax
from jax.experimental import pallas as pl
from jax.experimental.pallas import tpu as pltpu
from jax.experimental.pallas import tpu_sc as plsc

N = 10000
E = 320000
D = 128
H = 32
C = 16

NC = 2
NS = 16
NW = NC * NS
CH = 128
NCHUNK = 80
EPW = CH * NCHUNK
EPAD = NW * EPW
NPAD = 10240
RPT = NPAD // NS


def _make_spmm(F):
    fv = F // 16
    mesh = plsc.VectorSubcoreMesh(
        core_axis_name="c", subcore_axis_name="s", num_cores=NC, num_subcores=NS
    )

    @functools.partial(
        pl.kernel,
        out_type=jax.ShapeDtypeStruct((NC * NPAD, F), jnp.float32),
        mesh=mesh,
        compiler_params=pltpu.CompilerParams(use_tc_tiling_on_sc=False),
        scratch_types=[
            pltpu.VMEM((NCHUNK, CH), jnp.int32),
            pltpu.VMEM((NCHUNK, CH), jnp.int32),
            pltpu.VMEM((NCHUNK, CH), jnp.float32),
            pltpu.VMEM((CH, F), jnp.float32),
            pltpu.VMEM((CH, F), jnp.float32),
            pltpu.VMEM_SHARED((NPAD, F), jnp.float32),
            pltpu.SemaphoreType.DMA,
            pltpu.SemaphoreType.DMA,
        ],
    )
    def spmm(src_hbm, dst_hbm, ew_hbm, sup_hbm, out_hbm, src_v, dst_v, ew_v,
             gb0, gb1, acc_sh, sem0, sem1):
        cid = lax.axis_index("c")
        sid = lax.axis_index("s")
        wid = cid * NS + sid
        gbufs = (gb0, gb1)
        sems = (sem0, sem1)

        pltpu.sync_copy(src_hbm.at[wid], src_v)
        pltpu.sync_copy(dst_hbm.at[wid], dst_v)
        pltpu.sync_copy(ew_hbm.at[wid], ew_v)

        def zero_row(e, carry):
            for k in range(fv):
                gb0[e, pl.ds(16 * k, 16)] = jnp.zeros((16,), jnp.float32)
            return carry

        lax.fori_loop(0, CH, zero_row, 0, unroll=4)
        for r in range(RPT // CH):
            pltpu.sync_copy(gb0, acc_sh.at[pl.ds(sid * RPT + r * CH, CH)])
        plsc.subcore_barrier()

        def gather(j, b):
            return pltpu.async_copy(sup_hbm.at[src_v.at[j]], gbufs[b],
                                    sems[b])

        def gather_wait(j, b):
            pltpu.make_async_copy(sup_hbm.at[src_v.at[j]], gbufs[b],
                                  sems[b]).wait()

        def scale_scatter(j, b):
            gb = gbufs[b]
            for g in range(CH // 16):
                wv = ew_v[j, pl.ds(16 * g, 16)]
                for l in range(16):
                    w = wv[l]
                    e = 16 * g + l
                    for k in range(fv):
                        gb[e, pl.ds(16 * k, 16)] = (
                            gb[e, pl.ds(16 * k, 16)] * w
                        )
            pltpu.sync_copy(gb, acc_sh.at[dst_v.at[j]], add=True)

        gather(0, 0)
        gather(1, 1)

        def body(jj, carry):
            for b in range(2):
                j = 2 * jj + b
                gather_wait(j, b)
                scale_scatter(j, b)
                gather(j + 2, b)
            return carry

        lax.fori_loop(0, NCHUNK // 2 - 1, body, 0)
        for b in range(2):
            j = NCHUNK - 2 + b
            gather_wait(j, b)
            scale_scatter(j, b)

        plsc.subcore_barrier()
        pltpu.sync_copy(
            acc_sh.at[pl.ds(sid * RPT, RPT)],
            out_hbm.at[pl.ds(cid * NPAD + sid * RPT, RPT)],
        )

    return spmm


_spmm64 = _make_spmm(64)
_spmm32 = _make_spmm(32)

_RB = 1000


def _tc1_body(x0_ref, x1_ref, w10_ref, w11_ref, out_ref):
    s0 = jnp.dot(x0_ref[...], w10_ref[...], preferred_element_type=jnp.float32)
    s1 = jnp.dot(x1_ref[...], w11_ref[...], preferred_element_type=jnp.float32)
    out_ref[...] = jnp.concatenate([s0, s1], axis=1)


def _tc1(x0, x1, W1_0, W1_1):
    return pl.pallas_call(
        _tc1_body,
        grid=(N // _RB,),
        in_specs=[
            pl.BlockSpec((_RB, D), lambda i: (i, 0)),
            pl.BlockSpec((_RB, D), lambda i: (i, 0)),
            pl.BlockSpec((D, H), lambda i: (0, 0)),
            pl.BlockSpec((D, H), lambda i: (0, 0)),
        ],
        out_specs=pl.BlockSpec((_RB, 2 * H), lambda i: (i, 0)),
        out_shape=jax.ShapeDtypeStruct((N, 2 * H), jnp.float32),
    )(x0, x1, W1_0, W1_1)


def _tc2_body(p0_ref, p1_ref, b10_ref, b11_ref, w20_ref, w21_ref, out_ref):
    agg = p0_ref[...] + p1_ref[...]
    h0 = jax.nn.relu(agg[:, :H] + b10_ref[...])
    h1 = jax.nn.relu(agg[:, H:] + b11_ref[...])
    o0 = jnp.dot(h0, w20_ref[...], preferred_element_type=jnp.float32)
    o1 = jnp.dot(h1, w21_ref[...], preferred_element_type=jnp.float32)
    out_ref[...] = jnp.concatenate([o0, o1], axis=1)


def _tc2(p0, p1, b1_0, b1_1, W2_0, W2_1):
    return pl.pallas_call(
        _tc2_body,
        grid=(N // _RB,),
        in_specs=[
            pl.BlockSpec((_RB, 2 * H), lambda i: (i, 0)),
            pl.BlockSpec((_RB, 2 * H), lambda i: (i, 0)),
            pl.BlockSpec((1, H), lambda i: (0, 0)),
            pl.BlockSpec((1, H), lambda i: (0, 0)),
            pl.BlockSpec((H, C), lambda i: (0, 0)),
            pl.BlockSpec((H, C), lambda i: (0, 0)),
        ],
        out_specs=pl.BlockSpec((_RB, 2 * C), lambda i: (i, 0)),
        out_shape=jax.ShapeDtypeStruct((N, 2 * C), jnp.float32),
    )(p0, p1, b1_0.reshape(1, H), b1_1.reshape(1, H), W2_0, W2_1)


def _log_softmax(x):
    m = jnp.max(x, axis=1, keepdims=True)
    s = x - m
    return s - jnp.log(jnp.sum(jnp.exp(s), axis=1, keepdims=True))


def _tc3_body(p0_ref, p1_ref, b20_ref, b21_ref, a1_ref, a1b_ref, a2_ref,
              a3_ref, out_ref):
    agg = p0_ref[...] + p1_ref[...]
    z0 = _log_softmax(agg[:, :C] + b20_ref[...])
    z1 = _log_softmax(agg[:, C:] + b21_ref[...])
    a1 = a1_ref[...]
    a1b = a1b_ref[...]
    a2 = a2_ref[...]
    a3 = a3_ref[...]
    h0 = jnp.tanh(jnp.dot(z0, a1, preferred_element_type=jnp.float32) + a1b)
    h1 = jnp.tanh(jnp.dot(z1, a1, preferred_element_type=jnp.float32) + a1b)
    g0 = jnp.tanh(jnp.dot(h0, a2, preferred_element_type=jnp.float32))
    g1 = jnp.tanh(jnp.dot(h1, a2, preferred_element_type=jnp.float32))
    w0 = jnp.sum(g0 * a3, axis=1, keepdims=True)
    w1 = jnp.sum(g1 * a3, axis=1, keepdims=True)
    m = jnp.maximum(w0, w1)
    s0 = jnp.exp(w0 - m)
    s1 = jnp.exp(w1 - m)
    out_ref[...] = (s0 * z0 + s1 * z1) / (s0 + s1)


def _tc3(p0, p1, b2_0, b2_1, A1, a1b, A2, A3):
    FH = A1.shape[1]
    return pl.pallas_call(
        _tc3_body,
        grid=(N // _RB,),
        in_specs=[
            pl.BlockSpec((_RB, 2 * C), lambda i: (i, 0)),
            pl.BlockSpec((_RB, 2 * C), lambda i: (i, 0)),
            pl.BlockSpec((1, C), lambda i: (0, 0)),
            pl.BlockSpec((1, C), lambda i: (0, 0)),
            pl.BlockSpec((C, FH), lambda i: (0, 0)),
            pl.BlockSpec((1, FH), lambda i: (0, 0)),
            pl.BlockSpec((FH, 32), lambda i: (0, 0)),
            pl.BlockSpec((1, 32), lambda i: (0, 0)),
        ],
        out_specs=pl.BlockSpec((_RB, C), lambda i: (i, 0)),
        out_shape=jax.ShapeDtypeStruct((N, C), jnp.float32),
    )(p0, p1, b2_0.reshape(1, C), b2_1.reshape(1, C), A1,
      a1b.reshape(1, FH), A2, A3.reshape(1, 32))


def kernel(x0, x1, edge_index, edge_weight, W1_0, b1_0, W2_0, b2_0, W1_1,
           b1_1, W2_1, b2_1, A1, a1b, A2, A3):
    pad = EPAD - E
    src = jnp.concatenate(
        [edge_index[1].astype(jnp.int32), jnp.zeros((pad,), jnp.int32)]
    ).reshape(NW, NCHUNK, CH)
    dst = jnp.concatenate(
        [edge_index[0].astype(jnp.int32), jnp.zeros((pad,), jnp.int32)]
    ).reshape(NW, NCHUNK, CH)
    ew = jnp.concatenate(
        [edge_weight.astype(jnp.float32), jnp.zeros((pad,), jnp.float32)]
    ).reshape(NW, NCHUNK, CH)

    sup = _tc1(x0, x1, W1_0, W1_1)
    parts = _spmm64(src, dst, ew, sup)
    h = _tc2(parts[:N], parts[NPAD:NPAD + N], b1_0, b1_1, W2_0, W2_1)
    parts2 = _spmm32(src, dst, ew, h)
    return _tc3(parts2[:N], parts2[NPAD:NPAD + N], b2_0, b2_1, A1, a1b, A2,
                A3)


# speedup vs baseline: 18.0836x; 1.6773x over previous
import functools

import jax
import jax.numpy as jnp
from jax import lax
from jax.experimental import pallas as pl
from jax.experimental.pallas import tpu as pltpu
from jax.experimental.pallas import tpu_sc as plsc

N = 10000
E = 320000
D = 128
H = 32
C = 16

NC = 2
NS = 16
NW = NC * NS
CH = 128
NCHUNK = 80
EPW = CH * NCHUNK
EPAD = NW * EPW
NPAD = 10240
RPT = NPAD // NS


def _make_spmm(F):
    fv = F // 16
    mesh = plsc.VectorSubcoreMesh(
        core_axis_name="c", subcore_axis_name="s", num_cores=NC, num_subcores=NS
    )

    @functools.partial(
        pl.kernel,
        out_type=jax.ShapeDtypeStruct((NC * NPAD, F), jnp.float32),
        mesh=mesh,
        compiler_params=pltpu.CompilerParams(use_tc_tiling_on_sc=False),
        scratch_types=[
            pltpu.VMEM((NCHUNK, CH), jnp.int32),
            pltpu.VMEM((NCHUNK, CH), jnp.int32),
            pltpu.VMEM((NCHUNK, CH), jnp.float32),
            pltpu.VMEM((CH, F), jnp.float32),
            pltpu.VMEM((CH, F), jnp.float32),
            pltpu.VMEM_SHARED((NPAD, F), jnp.float32),
            pltpu.VMEM_SHARED((N, F), jnp.float32),
            pltpu.SemaphoreType.DMA,
            pltpu.SemaphoreType.DMA,
        ],
    )
    def spmm(src_hbm, dst_hbm, ew_hbm, sup_hbm, out_hbm, src_v, dst_v, ew_v,
             gb0, gb1, acc_sh, tab_sh, sem0, sem1):
        cid = lax.axis_index("c")
        sid = lax.axis_index("s")
        wid = cid * NS + sid
        gbufs = (gb0, gb1)
        sems = (sem0, sem1)

        pltpu.sync_copy(src_hbm.at[wid], src_v)
        pltpu.sync_copy(dst_hbm.at[wid], dst_v)
        pltpu.sync_copy(ew_hbm.at[wid], ew_v)

        tr = N // NS
        pltpu.sync_copy(sup_hbm.at[pl.ds(sid * tr, tr)],
                        tab_sh.at[pl.ds(sid * tr, tr)])

        def zero_row(e, carry):
            for k in range(fv):
                gb0[e, pl.ds(16 * k, 16)] = jnp.zeros((16,), jnp.float32)
            return carry

        lax.fori_loop(0, CH, zero_row, 0, unroll=4)
        for r in range(RPT // CH):
            pltpu.sync_copy(gb0, acc_sh.at[pl.ds(sid * RPT + r * CH, CH)])
        plsc.subcore_barrier()

        def gather(j, b):
            return pltpu.async_copy(tab_sh.at[src_v.at[j]], gbufs[b],
                                    sems[b])

        def gather_wait(j, b):
            pltpu.make_async_copy(tab_sh.at[src_v.at[j]], gbufs[b],
                                  sems[b]).wait()

        def scale_scatter(j, b):
            gb = gbufs[b]
            for g in range(CH // 16):
                wv = ew_v[j, pl.ds(16 * g, 16)]
                for l in range(16):
                    w = wv[l]
                    e = 16 * g + l
                    for k in range(fv):
                        gb[e, pl.ds(16 * k, 16)] = (
                            gb[e, pl.ds(16 * k, 16)] * w
                        )
            pltpu.sync_copy(gb, acc_sh.at[dst_v.at[j]], add=True)

        gather(0, 0)
        gather(1, 1)

        def body(jj, carry):
            for b in range(2):
                j = 2 * jj + b
                gather_wait(j, b)
                scale_scatter(j, b)
                gather(j + 2, b)
            return carry

        lax.fori_loop(0, NCHUNK // 2 - 1, body, 0)
        for b in range(2):
            j = NCHUNK - 2 + b
            gather_wait(j, b)
            scale_scatter(j, b)

        plsc.subcore_barrier()
        pltpu.sync_copy(
            acc_sh.at[pl.ds(sid * RPT, RPT)],
            out_hbm.at[pl.ds(cid * NPAD + sid * RPT, RPT)],
        )

    return spmm


_spmm64 = _make_spmm(64)
_spmm32 = _make_spmm(32)

_RB = 1000


def _tc1_body(x0_ref, x1_ref, w10_ref, w11_ref, out_ref):
    s0 = jnp.dot(x0_ref[...], w10_ref[...], preferred_element_type=jnp.float32)
    s1 = jnp.dot(x1_ref[...], w11_ref[...], preferred_element_type=jnp.float32)
    out_ref[...] = jnp.concatenate([s0, s1], axis=1)


def _tc1(x0, x1, W1_0, W1_1):
    return pl.pallas_call(
        _tc1_body,
        grid=(N // _RB,),
        in_specs=[
            pl.BlockSpec((_RB, D), lambda i: (i, 0)),
            pl.BlockSpec((_RB, D), lambda i: (i, 0)),
            pl.BlockSpec((D, H), lambda i: (0, 0)),
            pl.BlockSpec((D, H), lambda i: (0, 0)),
        ],
        out_specs=pl.BlockSpec((_RB, 2 * H), lambda i: (i, 0)),
        out_shape=jax.ShapeDtypeStruct((N, 2 * H), jnp.float32),
    )(x0, x1, W1_0, W1_1)


def _tc2_body(p0_ref, p1_ref, b10_ref, b11_ref, w20_ref, w21_ref, out_ref):
    agg = p0_ref[...] + p1_ref[...]
    h0 = jax.nn.relu(agg[:, :H] + b10_ref[...])
    h1 = jax.nn.relu(agg[:, H:] + b11_ref[...])
    o0 = jnp.dot(h0, w20_ref[...], preferred_element_type=jnp.float32)
    o1 = jnp.dot(h1, w21_ref[...], preferred_element_type=jnp.float32)
    out_ref[...] = jnp.concatenate([o0, o1], axis=1)


def _tc2(p0, p1, b1_0, b1_1, W2_0, W2_1):
    return pl.pallas_call(
        _tc2_body,
        grid=(N // _RB,),
        in_specs=[
            pl.BlockSpec((_RB, 2 * H), lambda i: (i, 0)),
            pl.BlockSpec((_RB, 2 * H), lambda i: (i, 0)),
            pl.BlockSpec((1, H), lambda i: (0, 0)),
            pl.BlockSpec((1, H), lambda i: (0, 0)),
            pl.BlockSpec((H, C), lambda i: (0, 0)),
            pl.BlockSpec((H, C), lambda i: (0, 0)),
        ],
        out_specs=pl.BlockSpec((_RB, 2 * C), lambda i: (i, 0)),
        out_shape=jax.ShapeDtypeStruct((N, 2 * C), jnp.float32),
    )(p0, p1, b1_0.reshape(1, H), b1_1.reshape(1, H), W2_0, W2_1)


def _log_softmax(x):
    m = jnp.max(x, axis=1, keepdims=True)
    s = x - m
    return s - jnp.log(jnp.sum(jnp.exp(s), axis=1, keepdims=True))


def _tc3_body(p0_ref, p1_ref, b20_ref, b21_ref, a1_ref, a1b_ref, a2_ref,
              a3_ref, out_ref):
    agg = p0_ref[...] + p1_ref[...]
    z0 = _log_softmax(agg[:, :C] + b20_ref[...])
    z1 = _log_softmax(agg[:, C:] + b21_ref[...])
    a1 = a1_ref[...]
    a1b = a1b_ref[...]
    a2 = a2_ref[...]
    a3 = a3_ref[...]
    h0 = jnp.tanh(jnp.dot(z0, a1, preferred_element_type=jnp.float32) + a1b)
    h1 = jnp.tanh(jnp.dot(z1, a1, preferred_element_type=jnp.float32) + a1b)
    g0 = jnp.tanh(jnp.dot(h0, a2, preferred_element_type=jnp.float32))
    g1 = jnp.tanh(jnp.dot(h1, a2, preferred_element_type=jnp.float32))
    w0 = jnp.sum(g0 * a3, axis=1, keepdims=True)
    w1 = jnp.sum(g1 * a3, axis=1, keepdims=True)
    m = jnp.maximum(w0, w1)
    s0 = jnp.exp(w0 - m)
    s1 = jnp.exp(w1 - m)
    out_ref[...] = (s0 * z0 + s1 * z1) / (s0 + s1)


def _tc3(p0, p1, b2_0, b2_1, A1, a1b, A2, A3):
    FH = A1.shape[1]
    return pl.pallas_call(
        _tc3_body,
        grid=(N // _RB,),
        in_specs=[
            pl.BlockSpec((_RB, 2 * C), lambda i: (i, 0)),
            pl.BlockSpec((_RB, 2 * C), lambda i: (i, 0)),
            pl.BlockSpec((1, C), lambda i: (0, 0)),
            pl.BlockSpec((1, C), lambda i: (0, 0)),
            pl.BlockSpec((C, FH), lambda i: (0, 0)),
            pl.BlockSpec((1, FH), lambda i: (0, 0)),
            pl.BlockSpec((FH, 32), lambda i: (0, 0)),
            pl.BlockSpec((1, 32), lambda i: (0, 0)),
        ],
        out_specs=pl.BlockSpec((_RB, C), lambda i: (i, 0)),
        out_shape=jax.ShapeDtypeStruct((N, C), jnp.float32),
    )(p0, p1, b2_0.reshape(1, C), b2_1.reshape(1, C), A1,
      a1b.reshape(1, FH), A2, A3.reshape(1, 32))


def kernel(x0, x1, edge_index, edge_weight, W1_0, b1_0, W2_0, b2_0, W1_1,
           b1_1, W2_1, b2_1, A1, a1b, A2, A3):
    pad = EPAD - E
    src = jnp.concatenate(
        [edge_index[1].astype(jnp.int32), jnp.zeros((pad,), jnp.int32)]
    ).reshape(NW, NCHUNK, CH)
    dst = jnp.concatenate(
        [edge_index[0].astype(jnp.int32), jnp.zeros((pad,), jnp.int32)]
    ).reshape(NW, NCHUNK, CH)
    ew = jnp.concatenate(
        [edge_weight.astype(jnp.float32), jnp.zeros((pad,), jnp.float32)]
    ).reshape(NW, NCHUNK, CH)

    sup = _tc1(x0, x1, W1_0, W1_1)
    parts = _spmm64(src, dst, ew, sup)
    h = _tc2(parts[:N], parts[NPAD:NPAD + N], b1_0, b1_1, W2_0, W2_1)
    parts2 = _spmm32(src, dst, ew, h)
    return _tc3(parts2[:N], parts2[NPAD:NPAD + N], b2_0, b2_1, A1, a1b, A2,
                A3)


# speedup vs baseline: 23.8215x; 1.3173x over previous
import functools

import jax
import jax.numpy as jnp
from jax import lax
from jax.experimental import pallas as pl
from jax.experimental.pallas import tpu as pltpu
from jax.experimental.pallas import tpu_sc as plsc

N = 10000
E = 320000
D = 128
H = 32
C = 16

NC = 2
NS = 16
NW = NC * NS
CH = 80
NCHUNK = 125
EPW = CH * NCHUNK
NPAD = 10240
RPT = NPAD // NS
TR = N // NS


def _make_spmm(F):
    fv = F // 16
    mesh = plsc.VectorSubcoreMesh(
        core_axis_name="c", subcore_axis_name="s", num_cores=NC, num_subcores=NS
    )

    @functools.partial(
        pl.kernel,
        out_type=jax.ShapeDtypeStruct((NC * NPAD, F), jnp.float32),
        mesh=mesh,
        compiler_params=pltpu.CompilerParams(use_tc_tiling_on_sc=False),
        scratch_types=[
            pltpu.VMEM((NCHUNK, CH), jnp.int32),
            pltpu.VMEM((NCHUNK, CH), jnp.int32),
            pltpu.VMEM((NCHUNK, CH), jnp.float32),
            pltpu.VMEM((CH, F), jnp.float32),
            pltpu.VMEM((CH, F), jnp.float32),
            pltpu.VMEM_SHARED((NPAD, F), jnp.float32),
            pltpu.VMEM_SHARED((N, F), jnp.float32),
            pltpu.SemaphoreType.DMA,
            pltpu.SemaphoreType.DMA,
        ],
    )
    def spmm(src_hbm, dst_hbm, ew_hbm, sup_hbm, out_hbm, src_v, dst_v, ew_v,
             gb0, gb1, acc_sh, tab_sh, sem0, sem1):
        cid = lax.axis_index("c")
        sid = lax.axis_index("s")
        wid = cid * NS + sid
        gbufs = (gb0, gb1)
        sems = (sem0, sem1)

        pltpu.sync_copy(src_hbm.at[wid], src_v)
        pltpu.sync_copy(dst_hbm.at[wid], dst_v)
        pltpu.sync_copy(ew_hbm.at[wid], ew_v)

        pltpu.sync_copy(sup_hbm.at[pl.ds(sid * TR, TR)],
                        tab_sh.at[pl.ds(sid * TR, TR)])

        def zero_row(e, carry):
            for k in range(fv):
                gb0[e, pl.ds(16 * k, 16)] = jnp.zeros((16,), jnp.float32)
            return carry

        lax.fori_loop(0, CH, zero_row, 0, unroll=4)
        for r in range(RPT // CH):
            pltpu.sync_copy(gb0, acc_sh.at[pl.ds(sid * RPT + r * CH, CH)])
        plsc.subcore_barrier()

        def gather(j, b):
            return pltpu.async_copy(tab_sh.at[src_v.at[j]], gbufs[b],
                                    sems[b])

        def gather_wait(j, b):
            pltpu.make_async_copy(tab_sh.at[src_v.at[j]], gbufs[b],
                                  sems[b]).wait()

        def scale_scatter(j, b):
            gb = gbufs[b]
            for g in range(CH // 16):
                wv = ew_v[j, pl.ds(16 * g, 16)]
                for l in range(16):
                    w = wv[l]
                    e = 16 * g + l
                    for k in range(fv):
                        gb[e, pl.ds(16 * k, 16)] = (
                            gb[e, pl.ds(16 * k, 16)] * w
                        )
            pltpu.sync_copy(gb, acc_sh.at[dst_v.at[j]], add=True)

        gather(0, 0)
        gather(1, 1)

        def body(jj, carry):
            for b in range(2):
                j = 2 * jj + b
                gather_wait(j, b)
                scale_scatter(j, b)
                gather(j + 2, b)
            return carry

        lax.fori_loop(0, NCHUNK // 2 - 1, body, 0)
        gather_wait(NCHUNK - 3, 0)
        scale_scatter(NCHUNK - 3, 0)
        gather(NCHUNK - 1, 0)
        gather_wait(NCHUNK - 2, 1)
        scale_scatter(NCHUNK - 2, 1)
        gather_wait(NCHUNK - 1, 0)
        scale_scatter(NCHUNK - 1, 0)

        plsc.subcore_barrier()
        pltpu.sync_copy(
            acc_sh.at[pl.ds(sid * RPT, RPT)],
            out_hbm.at[pl.ds(cid * NPAD + sid * RPT, RPT)],
        )

    return spmm


_spmm64 = _make_spmm(64)
_spmm32 = _make_spmm(32)

_RB1 = 1000
_RB = 1024


def _tc1_body(x0_ref, x1_ref, w10_ref, w11_ref, out_ref):
    s0 = jnp.dot(x0_ref[...], w10_ref[...], preferred_element_type=jnp.float32)
    s1 = jnp.dot(x1_ref[...], w11_ref[...], preferred_element_type=jnp.float32)
    out_ref[...] = jnp.concatenate([s0, s1], axis=1)


def _tc1(x0, x1, W1_0, W1_1):
    return pl.pallas_call(
        _tc1_body,
        grid=(N // _RB1,),
        in_specs=[
            pl.BlockSpec((_RB1, D), lambda i: (i, 0)),
            pl.BlockSpec((_RB1, D), lambda i: (i, 0)),
            pl.BlockSpec((D, H), lambda i: (0, 0)),
            pl.BlockSpec((D, H), lambda i: (0, 0)),
        ],
        out_specs=pl.BlockSpec((_RB1, 2 * H), lambda i: (i, 0)),
        out_shape=jax.ShapeDtypeStruct((N, 2 * H), jnp.float32),
    )(x0, x1, W1_0, W1_1)


def _tc2_body(p0_ref, p1_ref, b10_ref, b11_ref, w20_ref, w21_ref, out_ref):
    agg = p0_ref[...] + p1_ref[...]
    h0 = jax.nn.relu(agg[:, :H] + b10_ref[...])
    h1 = jax.nn.relu(agg[:, H:] + b11_ref[...])
    o0 = jnp.dot(h0, w20_ref[...], preferred_element_type=jnp.float32)
    o1 = jnp.dot(h1, w21_ref[...], preferred_element_type=jnp.float32)
    out_ref[...] = jnp.concatenate([o0, o1], axis=1)


def _tc2(parts, b1_0, b1_1, W2_0, W2_1):
    nb = NPAD // _RB
    return pl.pallas_call(
        _tc2_body,
        grid=(nb,),
        in_specs=[
            pl.BlockSpec((_RB, 2 * H), lambda i: (i, 0)),
            pl.BlockSpec((_RB, 2 * H), lambda i, _nb=nb: (i + _nb, 0)),
            pl.BlockSpec((1, H), lambda i: (0, 0)),
            pl.BlockSpec((1, H), lambda i: (0, 0)),
            pl.BlockSpec((H, C), lambda i: (0, 0)),
            pl.BlockSpec((H, C), lambda i: (0, 0)),
        ],
        out_specs=pl.BlockSpec((_RB, 2 * C), lambda i: (i, 0)),
        out_shape=jax.ShapeDtypeStruct((NPAD, 2 * C), jnp.float32),
    )(parts, parts, b1_0.reshape(1, H), b1_1.reshape(1, H), W2_0, W2_1)


def _log_softmax(x):
    m = jnp.max(x, axis=1, keepdims=True)
    s = x - m
    return s - jnp.log(jnp.sum(jnp.exp(s), axis=1, keepdims=True))


def _tc3_body(p0_ref, p1_ref, b2_ref, a1d_ref, a1b2_ref, a2d_ref, a3r_ref,
              out_ref):
    agg = p0_ref[...] + p1_ref[...] + b2_ref[...]
    z0 = _log_softmax(agg[:, :C])
    z1 = _log_softmax(agg[:, C:])
    zcat = jnp.concatenate([z0, z1], axis=1)
    h = jnp.tanh(
        jnp.dot(zcat, a1d_ref[...], preferred_element_type=jnp.float32)
        + a1b2_ref[...]
    )
    g = jnp.tanh(jnp.dot(h, a2d_ref[...], preferred_element_type=jnp.float32))
    a3r = a3r_ref[...]
    w0 = jnp.sum(g[:, :32] * a3r, axis=1, keepdims=True)
    w1 = jnp.sum(g[:, 32:] * a3r, axis=1, keepdims=True)
    m = jnp.maximum(w0, w1)
    s0 = jnp.exp(w0 - m)
    s1 = jnp.exp(w1 - m)
    out_ref[...] = (s0 * z0 + s1 * z1) / (s0 + s1)


def _tc3(parts2, b2_0, b2_1, A1, a1b, A2, A3):
    FH = A1.shape[1]
    a1d = jnp.zeros((2 * C, 2 * FH), jnp.float32)
    a1d = a1d.at[:C, :FH].set(A1).at[C:, FH:].set(A1)
    a2d = jnp.zeros((2 * FH, 64), jnp.float32)
    a2d = a2d.at[:FH, :32].set(A2).at[FH:, 32:].set(A2)
    a1b2 = jnp.concatenate([a1b, a1b]).reshape(1, 2 * FH)
    b2 = jnp.concatenate([b2_0, b2_1]).reshape(1, 2 * C)
    a3r = A3.reshape(1, 32)
    nb = NPAD // _RB
    return pl.pallas_call(
        _tc3_body,
        grid=(nb,),
        in_specs=[
            pl.BlockSpec((_RB, 2 * C), lambda i: (i, 0)),
            pl.BlockSpec((_RB, 2 * C), lambda i, _nb=nb: (i + _nb, 0)),
            pl.BlockSpec((1, 2 * C), lambda i: (0, 0)),
            pl.BlockSpec((2 * C, 2 * FH), lambda i: (0, 0)),
            pl.BlockSpec((1, 2 * FH), lambda i: (0, 0)),
            pl.BlockSpec((2 * FH, 64), lambda i: (0, 0)),
            pl.BlockSpec((1, 32), lambda i: (0, 0)),
        ],
        out_specs=pl.BlockSpec((_RB, C), lambda i: (i, 0)),
        out_shape=jax.ShapeDtypeStruct((NPAD, C), jnp.float32),
    )(parts2, parts2, b2, a1d, a1b2, a2d, a3r)


def kernel(x0, x1, edge_index, edge_weight, W1_0, b1_0, W2_0, b2_0, W1_1,
           b1_1, W2_1, b2_1, A1, a1b, A2, A3):
    src = edge_index[1].astype(jnp.int32).reshape(NW, NCHUNK, CH)
    dst = edge_index[0].astype(jnp.int32).reshape(NW, NCHUNK, CH)
    ew = edge_weight.astype(jnp.float32).reshape(NW, NCHUNK, CH)

    sup = _tc1(x0, x1, W1_0, W1_1)
    parts = _spmm64(src, dst, ew, sup)
    h = _tc2(parts, b1_0, b1_1, W2_0, W2_1)
    parts2 = _spmm32(src, dst, ew, h)
    out = _tc3(parts2, b2_0, b2_1, A1, a1b, A2, A3)
    return out[:N]
